# baseline (speedup 1.0000x reference)
import functools

import jax
import jax.numpy as jnp
from jax import lax
from jax.experimental import pallas as pl
from jax.experimental.pallas import tpu as pltpu
from jax.experimental.pallas import tpu_sc as plsc

B = 16384
D = 64
NC = 2
NS = 16
L = 16
NW = NC * NS
BPW = B // NW
G = BPW // L

_mesh = plsc.VectorSubcoreMesh(core_axis_name="c", subcore_axis_name="s")


@functools.partial(
    pl.kernel,
    mesh=_mesh,
    compiler_params=pltpu.CompilerParams(
        needs_layout_passes=False, use_tc_tiling_on_sc=False
    ),
    out_type=[
        jax.ShapeDtypeStruct((B,), jnp.float32),
        jax.ShapeDtypeStruct((B,), jnp.float32),
    ],
    scratch_types=[
        pltpu.VMEM((BPW,), jnp.int32),
        pltpu.VMEM((BPW,), jnp.int32),
        pltpu.VMEM((BPW,), jnp.int32),
        pltpu.VMEM((BPW, D), jnp.float32),
        pltpu.VMEM((BPW, D), jnp.float32),
        pltpu.VMEM((BPW, D), jnp.float32),
        pltpu.VMEM((BPW,), jnp.float32),
        pltpu.VMEM((BPW,), jnp.float32),
        pltpu.SemaphoreType.DMA,
    ],
)
def _bpr_sc(user_hbm, item_i_hbm, item_j_hbm, embed_user_hbm, embed_item_hbm,
            out_i_hbm, out_j_hbm,
            idx_u, idx_i, idx_j, rows_u, rows_i, rows_j, pred_i, pred_j, sem):
    wid = lax.axis_index("s") * NC + lax.axis_index("c")
    base = wid * BPW

    pltpu.sync_copy(user_hbm.at[pl.ds(base, BPW)], idx_u)
    pltpu.sync_copy(item_i_hbm.at[pl.ds(base, BPW)], idx_i)
    pltpu.sync_copy(item_j_hbm.at[pl.ds(base, BPW)], idx_j)

    cu = pltpu.async_copy(embed_user_hbm.at[idx_u], rows_u, sem)
    ci = pltpu.async_copy(embed_item_hbm.at[idx_i], rows_i, sem)
    cj = pltpu.async_copy(embed_item_hbm.at[idx_j], rows_j, sem)
    cu.wait()
    ci.wait()
    cj.wait()

    lane = lax.iota(jnp.int32, L)

    def group(g, carry):
        rg = g * L
        out_i = jnp.zeros((L,), jnp.float32)
        out_j = jnp.zeros((L,), jnp.float32)
        for k in range(L):
            r = rg + k
            acc_i = jnp.zeros((L,), jnp.float32)
            acc_j = jnp.zeros((L,), jnp.float32)
            for q in range(D // L):
                u = rows_u[r, pl.ds(q * L, L)]
                vi = rows_i[r, pl.ds(q * L, L)]
                vj = rows_j[r, pl.ds(q * L, L)]
                acc_i = acc_i + u * vi
                acc_j = acc_j + u * vj
            out_i = jnp.where(lane == k, jnp.sum(acc_i), out_i)
            out_j = jnp.where(lane == k, jnp.sum(acc_j), out_j)
        pred_i[pl.ds(rg, L)] = out_i
        pred_j[pl.ds(rg, L)] = out_j
        return carry

    lax.fori_loop(0, G, group, 0)

    pltpu.sync_copy(pred_i, out_i_hbm.at[pl.ds(base, BPW)])
    pltpu.sync_copy(pred_j, out_j_hbm.at[pl.ds(base, BPW)])


def kernel(user, item_i, item_j, embed_user, embed_item):
    out_i, out_j = _bpr_sc(user, item_i, item_j, embed_user, embed_item)
    return (out_i, out_j)


# speedup vs baseline: 1.5654x; 1.5654x over previous
import functools

import jax
import jax.numpy as jnp
from jax import lax
from jax.experimental import pallas as pl
from jax.experimental.pallas import tpu as pltpu
from jax.experimental.pallas import tpu_sc as plsc

B = 16384
D = 64
NC = 2
NS = 16
L = 16
NW = NC * NS
BPW = B // NW
C = 128
NCH = BPW // C

_mesh = plsc.VectorSubcoreMesh(core_axis_name="c", subcore_axis_name="s")


@functools.partial(
    pl.kernel,
    mesh=_mesh,
    compiler_params=pltpu.CompilerParams(
        needs_layout_passes=False, use_tc_tiling_on_sc=True
    ),
    out_type=[
        jax.ShapeDtypeStruct((B,), jnp.float32),
        jax.ShapeDtypeStruct((B,), jnp.float32),
    ],
    scratch_types=[
        pltpu.VMEM((BPW,), jnp.int32),
        pltpu.VMEM((BPW,), jnp.int32),
        pltpu.VMEM((BPW,), jnp.int32),
        pltpu.VMEM((C, D), jnp.float32),
        pltpu.VMEM((C, D), jnp.float32),
        pltpu.VMEM((C, D), jnp.float32),
        pltpu.VMEM((BPW,), jnp.float32),
        pltpu.VMEM((BPW,), jnp.float32),
        pltpu.SemaphoreType.DMA,
    ],
)
def _bpr_sc(user_hbm, item_i_hbm, item_j_hbm, eu_hbm, ei_hbm,
            out_i_hbm, out_j_hbm,
            sid_u, sid_i, sid_j,
            buf_u, buf_i, buf_j, pred_i, pred_j, sem):
    wid = lax.axis_index("s") * NC + lax.axis_index("c")
    base = wid * BPW

    pltpu.sync_copy(user_hbm.at[pl.ds(base, BPW)], sid_u)
    pltpu.sync_copy(item_i_hbm.at[pl.ds(base, BPW)], sid_i)
    pltpu.sync_copy(item_j_hbm.at[pl.ds(base, BPW)], sid_j)

    lane = lax.iota(jnp.int32, L)

    def chunk(c, carry):
        cb = c * C

        def fire(s, carry2):
            sb = s * L
            vu = sid_u[pl.ds(cb + sb, L)]
            vi = sid_i[pl.ds(cb + sb, L)]
            vj = sid_j[pl.ds(cb + sb, L)]
            for k in range(L):
                pltpu.make_async_copy(
                    eu_hbm.at[pl.ds(vu[k], 1), :],
                    buf_u.at[pl.ds(sb + k, 1), :], sem
                ).start()
                pltpu.make_async_copy(
                    ei_hbm.at[pl.ds(vi[k], 1), :],
                    buf_i.at[pl.ds(sb + k, 1), :], sem
                ).start()
                pltpu.make_async_copy(
                    ei_hbm.at[pl.ds(vj[k], 1), :],
                    buf_j.at[pl.ds(sb + k, 1), :], sem
                ).start()
            return carry2

        lax.fori_loop(0, C // L, fire, 0)
        pltpu.make_async_copy(eu_hbm.at[pl.ds(0, C), :], buf_u, sem).wait()
        pltpu.make_async_copy(eu_hbm.at[pl.ds(0, C), :], buf_i, sem).wait()
        pltpu.make_async_copy(eu_hbm.at[pl.ds(0, C), :], buf_j, sem).wait()

        def group(g, carry2):
            rg = g * L
            out_i = jnp.zeros((L,), jnp.float32)
            out_j = jnp.zeros((L,), jnp.float32)
            for k in range(L):
                acc_i = jnp.zeros((L,), jnp.float32)
                acc_j = jnp.zeros((L,), jnp.float32)
                for q in range(D // L):
                    u = buf_u[rg + k, pl.ds(q * L, L)]
                    vi = buf_i[rg + k, pl.ds(q * L, L)]
                    vj = buf_j[rg + k, pl.ds(q * L, L)]
                    acc_i = acc_i + u * vi
                    acc_j = acc_j + u * vj
                out_i = jnp.where(lane == k, jnp.sum(acc_i), out_i)
                out_j = jnp.where(lane == k, jnp.sum(acc_j), out_j)
            pred_i[pl.ds(cb + rg, L)] = out_i
            pred_j[pl.ds(cb + rg, L)] = out_j
            return carry2

        lax.fori_loop(0, C // L, group, 0)
        return carry

    lax.fori_loop(0, NCH, chunk, 0)

    pltpu.sync_copy(pred_i, out_i_hbm.at[pl.ds(base, BPW)])
    pltpu.sync_copy(pred_j, out_j_hbm.at[pl.ds(base, BPW)])


def kernel(user, item_i, item_j, embed_user, embed_item):
    out_i, out_j = _bpr_sc(user, item_i, item_j, embed_user, embed_item)
    return (out_i, out_j)
